# Initial kernel scaffold; baseline (speedup 1.0000x reference)
import jax
import jax.numpy as jnp
from jax.experimental import pallas as pl


def kernel(x, rel_emb, W_neigh1, W_self1, W_neigh2, W_self2, edge_index, edge_type):
    raise NotImplementedError("write your pallas kernel here")


# speedup vs baseline: 3.6867x; 3.6867x over previous
import functools

import jax
import jax.numpy as jnp
from jax import lax
from jax.experimental import pallas as pl
from jax.experimental.pallas import tpu as pltpu
from jax.experimental.pallas import tpu_sc as plsc

N_NODES = 10000
N_EDGES = 320000
H_DIM = 128
N_REL = 200

NC, NS, L = 2, 16, 16
NODES_PAD = 10112
T_ROWS = NODES_PAD + N_REL
CHUNK = 128
ITEMS_PER_TEC = 20096
HALF_ITEMS = NS * ITEMS_PER_TEC
N_CHUNKS = ITEMS_PER_TEC // CHUNK
ROWS_PER_TEC = NODES_PAD // NS
DUMMY_DST = N_NODES
SLOPE = (1.0 / 8.0 + 1.0 / 3.0) / 2.0


def _prep_body(x_ref, wn_ref, ws_ref, xw_ref, hs_ref):
    xb = x_ref[...]
    xw_ref[...] = jnp.dot(xb, wn_ref[...], preferred_element_type=jnp.float32)
    hs_ref[...] = jnp.dot(xb, ws_ref[...], preferred_element_type=jnp.float32)


def _rel_mm_body(r_ref, et_ref, w_ref, o_ref, eto_ref):
    o_ref[...] = jnp.dot(r_ref[...], w_ref[...], preferred_element_type=jnp.float32)
    eto_ref[...] = et_ref[...] + NODES_PAD


def _combine_mm_body(a0_ref, a1_ref, d0_ref, d1_ref, hs_ref, wn_ref, ws_ref,
                     hw_ref, hs2_ref):
    h = _combined_h(a0_ref, a1_ref, d0_ref, d1_ref, hs_ref)
    hw_ref[...] = jnp.dot(h, wn_ref[...], preferred_element_type=jnp.float32)
    hs2_ref[...] = jnp.dot(h, ws_ref[...], preferred_element_type=jnp.float32)


def _combine_out_body(a0_ref, a1_ref, d0_ref, d1_ref, hs_ref, h_ref):
    h_ref[...] = _combined_h(a0_ref, a1_ref, d0_ref, d1_ref, hs_ref)


def _combined_h(a0_ref, a1_ref, d0_ref, d1_ref, hs_ref):
    agg = a0_ref[...] + a1_ref[...]
    deg = d0_ref[...][:, 0:1] + d1_ref[...][:, 0:1]
    deg = jnp.clip(deg, 1.0, None)
    h = agg / deg + hs_ref[...]
    h = jnp.where(h >= 0, h, h * SLOPE)
    norm = jnp.sqrt(jnp.sum(h * h, axis=1, keepdims=True))
    return h / (norm + 1e-12)


_GRID = 8
_XB = NODES_PAD // _GRID


def _node_spec():
    return pl.BlockSpec((_XB, H_DIM), lambda i: (i, 0))


def _w_spec():
    return pl.BlockSpec((H_DIM, H_DIM), lambda i: (0, 0))


def _prep_call(x, wn, ws):
    return pl.pallas_call(
        _prep_body,
        grid=(_GRID,),
        in_specs=[_node_spec(), _w_spec(), _w_spec()],
        out_specs=[_node_spec(), _node_spec()],
        out_shape=[jax.ShapeDtypeStruct((NODES_PAD, H_DIM), jnp.float32),
                   jax.ShapeDtypeStruct((NODES_PAD, H_DIM), jnp.float32)],
    )(x, wn, ws)


def _rel_mm_call(rel, et2d, w):
    return pl.pallas_call(
        _rel_mm_body,
        out_shape=[jax.ShapeDtypeStruct((N_REL, H_DIM), jnp.float32),
                   jax.ShapeDtypeStruct(et2d.shape, jnp.int32)],
    )(rel, et2d, w)


def _deg_spec():
    return pl.BlockSpec((_XB, H_DIM), lambda i: (i, 0))


def _combine_mm_call(a0, a1, d0, d1, hs, wn, ws):
    return pl.pallas_call(
        _combine_mm_body,
        grid=(_GRID,),
        in_specs=[_node_spec(), _node_spec(), _deg_spec(), _deg_spec(),
                  _node_spec(), _w_spec(), _w_spec()],
        out_specs=[_node_spec(), _node_spec()],
        out_shape=[jax.ShapeDtypeStruct((NODES_PAD, H_DIM), jnp.float32),
                   jax.ShapeDtypeStruct((NODES_PAD, H_DIM), jnp.float32)],
    )(a0, a1, d0, d1, hs, wn, ws)


def _combine_out_call(a0, a1, d0, d1, hs):
    return pl.pallas_call(
        _combine_out_body,
        grid=(_GRID,),
        in_specs=[_node_spec(), _node_spec(), _deg_spec(), _deg_spec(),
                  _node_spec()],
        out_specs=_node_spec(),
        out_shape=jax.ShapeDtypeStruct((NODES_PAD, H_DIM), jnp.float32),
    )(a0, a1, d0, d1, hs)


def _mesh():
    return plsc.VectorSubcoreMesh(core_axis_name="c", subcore_axis_name="s",
                                  num_cores=NC, num_subcores=NS)


def _zero_shared(src_buf, shared, row0, step):
    zoff = 0
    while zoff < ROWS_PER_TEC:
        zn = min(step, ROWS_PER_TEC - zoff)
        pltpu.sync_copy(src_buf.at[pl.ds(0, zn)],
                        shared.at[pl.ds(row0 + zoff, zn)])
        zoff += zn


def _sc_agg_body(src2_hbm, dst2_hbm, t_hbm, agg_out,
                 sidx, didx, rows, agg_sp, gsem):
    c = lax.axis_index("c")
    s = lax.axis_index("s")

    def _zrow(r, carry):
        for j in range(H_DIM // L):
            rows[r, pl.ds(j * L, L)] = jnp.zeros((L,), jnp.float32)
        return carry
    lax.fori_loop(0, CHUNK, _zrow, 0)

    row0 = pl.multiple_of(s * ROWS_PER_TEC, ROWS_PER_TEC)
    _zero_shared(rows, agg_sp, row0, CHUNK)

    plsc.subcore_barrier()

    item0 = pl.multiple_of(c * HALF_ITEMS + s * ITEMS_PER_TEC, CHUNK)

    def _chunk(g, carry):
        base = pl.multiple_of(item0 + g * CHUNK, CHUNK)
        pltpu.sync_copy(src2_hbm.at[pl.ds(base, CHUNK)], sidx)
        pltpu.sync_copy(dst2_hbm.at[pl.ds(base, CHUNK)], didx)
        pltpu.async_copy(t_hbm.at[sidx], rows, gsem).wait()
        pltpu.sync_copy(rows, agg_sp.at[didx], add=True)
        return carry
    lax.fori_loop(0, N_CHUNKS, _chunk, 0)

    plsc.subcore_barrier()

    out_base = pl.multiple_of(c * NODES_PAD + row0, ROWS_PER_TEC)
    pltpu.sync_copy(agg_sp.at[pl.ds(row0, ROWS_PER_TEC)],
                    agg_out.at[pl.ds(out_base, ROWS_PER_TEC)])


def _make_sc_agg_kernel():
    scratch = [
        pltpu.VMEM((CHUNK,), jnp.int32),
        pltpu.VMEM((CHUNK,), jnp.int32),
        pltpu.VMEM((CHUNK, H_DIM), jnp.float32),
        pltpu.VMEM_SHARED((NODES_PAD, H_DIM), jnp.float32),
        pltpu.SemaphoreType.DMA,
    ]
    return pl.kernel(
        _sc_agg_body,
        out_type=[jax.ShapeDtypeStruct((NC * NODES_PAD, H_DIM), jnp.float32)],
        mesh=_mesh(), scratch_types=scratch)


DEG_CHUNK = 64
DEG_ITEMS_PER_TEC = HALF_ITEMS // (NC * NS)
DEG_CHUNKS = DEG_ITEMS_PER_TEC // DEG_CHUNK


def _sc_deg_body(dsth_hbm, deg_out, didx, ones, zbufd, deg_sp):
    c = lax.axis_index("c")
    s = lax.axis_index("s")

    def _fill(r, carry):
        for j in range(H_DIM // L):
            zbufd[r, pl.ds(j * L, L)] = jnp.zeros((L,), jnp.float32)
            ones[r, pl.ds(j * L, L)] = jnp.ones((L,), jnp.float32)
        return carry
    lax.fori_loop(0, DEG_CHUNK, _fill, 0)

    row0 = pl.multiple_of(s * ROWS_PER_TEC, ROWS_PER_TEC)
    _zero_shared(zbufd, deg_sp, row0, DEG_CHUNK)

    plsc.subcore_barrier()

    item0 = pl.multiple_of((c * NS + s) * DEG_ITEMS_PER_TEC, 8)

    def _chunk(g, carry):
        base = pl.multiple_of(item0 + g * DEG_CHUNK, 8)
        pltpu.sync_copy(dsth_hbm.at[pl.ds(base, DEG_CHUNK)], didx)
        pltpu.sync_copy(ones, deg_sp.at[didx], add=True)
        return carry
    lax.fori_loop(0, DEG_CHUNKS, _chunk, 0)

    plsc.subcore_barrier()

    out_base = pl.multiple_of(c * NODES_PAD + row0, ROWS_PER_TEC)
    pltpu.sync_copy(deg_sp.at[pl.ds(row0, ROWS_PER_TEC)],
                    deg_out.at[pl.ds(out_base, ROWS_PER_TEC)])


def _make_sc_deg_kernel():
    scratch = [
        pltpu.VMEM((DEG_CHUNK,), jnp.int32),
        pltpu.VMEM((DEG_CHUNK, H_DIM), jnp.float32),
        pltpu.VMEM((DEG_CHUNK, H_DIM), jnp.float32),
        pltpu.VMEM_SHARED((NODES_PAD, H_DIM), jnp.float32),
    ]
    return pl.kernel(
        _sc_deg_body,
        out_type=[jax.ShapeDtypeStruct((NC * NODES_PAD, H_DIM), jnp.float32)],
        mesh=_mesh(), scratch_types=scratch)


def kernel(x, rel_emb, W_neigh1, W_self1, W_neigh2, W_self2,
           edge_index, edge_type):
    src = edge_index[0].astype(jnp.int32)
    dst = edge_index[1].astype(jnp.int32)
    et = edge_type.astype(jnp.int32)

    x_pad = jnp.pad(x, ((0, NODES_PAD - N_NODES), (0, 0)))
    et2d = et.reshape(N_EDGES // H_DIM, H_DIM)

    xw, hs1 = _prep_call(x_pad, W_neigh1, W_self1)
    rw1, eto = _rel_mm_call(rel_emb, et2d, W_neigh1)

    pad_i = HALF_ITEMS - N_EDGES
    src_h = jnp.concatenate([src, jnp.zeros((pad_i,), jnp.int32)])
    rel_h = jnp.concatenate([eto.reshape(-1), jnp.zeros((pad_i,), jnp.int32)])
    src2 = jnp.concatenate([src_h, rel_h])
    dst_h = jnp.concatenate([dst, jnp.full((pad_i,), DUMMY_DST, jnp.int32)])
    dst2 = jnp.concatenate([dst_h, dst_h])

    (deg,) = _make_sc_deg_kernel()(dst_h)
    d0, d1 = deg[:NODES_PAD], deg[NODES_PAD:]

    t1 = jnp.concatenate([xw, rw1], axis=0)
    (agg1,) = _make_sc_agg_kernel()(src2, dst2, t1)
    a1_0, a1_1 = agg1[:NODES_PAD], agg1[NODES_PAD:]

    hw2, hs2 = _combine_mm_call(a1_0, a1_1, d0, d1, hs1, W_neigh2, W_self2)
    rw2, _ = _rel_mm_call(rel_emb, et2d, W_neigh2)

    t2 = jnp.concatenate([hw2, rw2], axis=0)
    (agg2,) = _make_sc_agg_kernel()(src2, dst2, t2)
    a2_0, a2_1 = agg2[:NODES_PAD], agg2[NODES_PAD:]

    h2 = _combine_out_call(a2_0, a2_1, d0, d1, hs2)
    return h2[:N_NODES]
